# baseline (speedup 1.0000x reference)
import jax
import jax.numpy as jnp
from jax import lax
from jax.experimental import pallas as pl
from jax.experimental.pallas import tpu as pltpu

D_MODEL = 768
NUM_EXPERTS = 64
N_TOKENS = 128
E_BLK = 2
N_STEPS = NUM_EXPERTS // E_BLK
NBUF = 4


def _moe_kernel(x_ref, gate_ref, w1_hbm, b1_hbm, w2_hbm, b2_hbm, out_ref,
                w1_bufs, w2_bufs, b1_bufs, b2_bufs, acc_ref, sems):
    x = x_ref[...]
    logits = jax.lax.dot_general(
        x, gate_ref[...], (((1,), (1,)), ((), ())),
        preferred_element_type=jnp.float32)
    eids = jax.lax.broadcasted_iota(jnp.int32, (N_TOKENS, NUM_EXPERTS), 1)
    big = jnp.int32(NUM_EXPERTS + 1)
    v1 = jnp.max(logits, axis=1, keepdims=True)
    i1 = jnp.min(jnp.where(logits == v1, eids, big), axis=1, keepdims=True)
    masked = jnp.where(eids == i1, -jnp.inf, logits)
    v2 = jnp.max(masked, axis=1, keepdims=True)
    i2 = jnp.min(jnp.where(masked == v2, eids, big), axis=1, keepdims=True)
    t = jnp.exp(v2 - v1)
    p1 = 1.0 / (1.0 + t)
    p2 = t / (1.0 + t)

    xb = x.astype(jnp.bfloat16)

    def _start(s, b):
        sl = pl.ds(s * E_BLK, E_BLK)
        pltpu.make_async_copy(w1_hbm.at[sl], w1_bufs.at[b],
                              sems.at[b, 0]).start()
        pltpu.make_async_copy(w2_hbm.at[sl], w2_bufs.at[b],
                              sems.at[b, 1]).start()
        pltpu.make_async_copy(b1_hbm.at[sl], b1_bufs.at[b],
                              sems.at[b, 2]).start()
        pltpu.make_async_copy(b2_hbm.at[sl], b2_bufs.at[b],
                              sems.at[b, 3]).start()

    def _wait(s, b):
        sl = pl.ds(s * E_BLK, E_BLK)
        pltpu.make_async_copy(w1_hbm.at[sl], w1_bufs.at[b],
                              sems.at[b, 0]).wait()
        pltpu.make_async_copy(w2_hbm.at[sl], w2_bufs.at[b],
                              sems.at[b, 1]).wait()
        pltpu.make_async_copy(b1_hbm.at[sl], b1_bufs.at[b],
                              sems.at[b, 2]).wait()
        pltpu.make_async_copy(b2_hbm.at[sl], b2_bufs.at[b],
                              sems.at[b, 3]).wait()

    for b in range(NBUF):
        _start(b, b)

    acc_ref[...] = jnp.zeros_like(acc_ref)

    def _body(s, carry):
        b = s % NBUF
        _wait(s, b)
        contrib = acc_ref[...]
        for k in range(E_BLK):
            e = s * E_BLK + k
            w1 = w1_bufs[b, k].astype(jnp.bfloat16)
            h = jax.lax.dot_general(xb, w1, (((1,), (0,)), ((), ())),
                                    preferred_element_type=jnp.float32)
            h += b1_bufs[b, k]
            h = h * 0.5 * (1.0 + jax.lax.erf(h * 0.7071067811865476))
            w2 = w2_bufs[b, k].astype(jnp.bfloat16)
            o = jax.lax.dot_general(h.astype(jnp.bfloat16), w2,
                                    (((1,), (0,)), ((), ())),
                                    preferred_element_type=jnp.float32)
            o += b2_bufs[b, k]
            wcol = jnp.where(i1 == e, p1, 0.0) + jnp.where(i2 == e, p2, 0.0)
            contrib += o * wcol
        acc_ref[...] = contrib

        @pl.when(s + NBUF < N_STEPS)
        def _rearm():
            _start(s + NBUF, b)

        return carry

    lax.fori_loop(0, N_STEPS, _body, jnp.int32(0))
    out_ref[...] = acc_ref[...]


@jax.jit
def kernel(x, gate_w, w1, b1, w2, b2):
    Bs, Ts, D = x.shape
    x_flat = x.reshape(-1, D)
    out = pl.pallas_call(
        _moe_kernel,
        in_specs=[
            pl.BlockSpec(memory_space=pltpu.VMEM),
            pl.BlockSpec(memory_space=pltpu.VMEM),
            pl.BlockSpec(memory_space=pl.ANY),
            pl.BlockSpec(memory_space=pl.ANY),
            pl.BlockSpec(memory_space=pl.ANY),
            pl.BlockSpec(memory_space=pl.ANY),
        ],
        out_shape=jax.ShapeDtypeStruct((N_TOKENS, D_MODEL), jnp.float32),
        scratch_shapes=[
            pltpu.VMEM((NBUF, E_BLK, D_MODEL, D_MODEL), jnp.float32),
            pltpu.VMEM((NBUF, E_BLK, D_MODEL, D_MODEL), jnp.float32),
            pltpu.VMEM((NBUF, E_BLK, D_MODEL), jnp.float32),
            pltpu.VMEM((NBUF, E_BLK, D_MODEL), jnp.float32),
            pltpu.VMEM((N_TOKENS, D_MODEL), jnp.float32),
            pltpu.SemaphoreType.DMA((NBUF, 4)),
        ],
    )(x_flat, gate_w, w1, b1, w2, b2)
    return out.reshape(Bs, Ts, D)


# speedup vs baseline: 1.0004x; 1.0004x over previous
import jax
import jax.numpy as jnp
from jax import lax
from jax.experimental import pallas as pl
from jax.experimental.pallas import tpu as pltpu

D_MODEL = 768
NUM_EXPERTS = 64
N_TOKENS = 128
E_BLK = 2
N_STEPS = NUM_EXPERTS // E_BLK
NBUF = 5


def _moe_kernel(x_ref, gate_ref, w1_hbm, b1_hbm, w2_hbm, b2_hbm, out_ref,
                w1_bufs, w2_bufs, b1_bufs, b2_bufs, acc_ref, sems):
    x = x_ref[...]
    logits = jax.lax.dot_general(
        x, gate_ref[...], (((1,), (1,)), ((), ())),
        preferred_element_type=jnp.float32)
    eids = jax.lax.broadcasted_iota(jnp.int32, (N_TOKENS, NUM_EXPERTS), 1)
    big = jnp.int32(NUM_EXPERTS + 1)
    v1 = jnp.max(logits, axis=1, keepdims=True)
    i1 = jnp.min(jnp.where(logits == v1, eids, big), axis=1, keepdims=True)
    masked = jnp.where(eids == i1, -jnp.inf, logits)
    v2 = jnp.max(masked, axis=1, keepdims=True)
    i2 = jnp.min(jnp.where(masked == v2, eids, big), axis=1, keepdims=True)
    t = jnp.exp(v2 - v1)
    p1 = 1.0 / (1.0 + t)
    p2 = t / (1.0 + t)

    xb = x.astype(jnp.bfloat16)

    def _start(s, b):
        sl = pl.ds(s * E_BLK, E_BLK)
        pltpu.make_async_copy(w1_hbm.at[sl], w1_bufs.at[b],
                              sems.at[b, 0]).start()
        pltpu.make_async_copy(w2_hbm.at[sl], w2_bufs.at[b],
                              sems.at[b, 1]).start()
        pltpu.make_async_copy(b1_hbm.at[sl], b1_bufs.at[b],
                              sems.at[b, 2]).start()
        pltpu.make_async_copy(b2_hbm.at[sl], b2_bufs.at[b],
                              sems.at[b, 3]).start()

    def _wait(s, b):
        sl = pl.ds(s * E_BLK, E_BLK)
        pltpu.make_async_copy(w1_hbm.at[sl], w1_bufs.at[b],
                              sems.at[b, 0]).wait()
        pltpu.make_async_copy(w2_hbm.at[sl], w2_bufs.at[b],
                              sems.at[b, 1]).wait()
        pltpu.make_async_copy(b1_hbm.at[sl], b1_bufs.at[b],
                              sems.at[b, 2]).wait()
        pltpu.make_async_copy(b2_hbm.at[sl], b2_bufs.at[b],
                              sems.at[b, 3]).wait()

    for b in range(NBUF):
        _start(b, b)

    acc_ref[...] = jnp.zeros_like(acc_ref)

    def _body(s, carry):
        b = s % NBUF
        _wait(s, b)
        contrib = acc_ref[...]
        for k in range(E_BLK):
            e = s * E_BLK + k
            w1 = w1_bufs[b, k].astype(jnp.bfloat16)
            h = jax.lax.dot_general(xb, w1, (((1,), (0,)), ((), ())),
                                    preferred_element_type=jnp.float32)
            h += b1_bufs[b, k]
            h = h * 0.5 * (1.0 + jax.lax.erf(h * 0.7071067811865476))
            w2 = w2_bufs[b, k].astype(jnp.bfloat16)
            o = jax.lax.dot_general(h.astype(jnp.bfloat16), w2,
                                    (((1,), (0,)), ((), ())),
                                    preferred_element_type=jnp.float32)
            o += b2_bufs[b, k]
            wcol = jnp.where(i1 == e, p1, 0.0) + jnp.where(i2 == e, p2, 0.0)
            contrib += o * wcol
        acc_ref[...] = contrib

        @pl.when(s + NBUF < N_STEPS)
        def _rearm():
            _start(s + NBUF, b)

        return carry

    lax.fori_loop(0, N_STEPS, _body, jnp.int32(0))
    out_ref[...] = acc_ref[...]


@jax.jit
def kernel(x, gate_w, w1, b1, w2, b2):
    Bs, Ts, D = x.shape
    x_flat = x.reshape(-1, D)
    out = pl.pallas_call(
        _moe_kernel,
        in_specs=[
            pl.BlockSpec(memory_space=pltpu.VMEM),
            pl.BlockSpec(memory_space=pltpu.VMEM),
            pl.BlockSpec(memory_space=pl.ANY),
            pl.BlockSpec(memory_space=pl.ANY),
            pl.BlockSpec(memory_space=pl.ANY),
            pl.BlockSpec(memory_space=pl.ANY),
        ],
        out_shape=jax.ShapeDtypeStruct((N_TOKENS, D_MODEL), jnp.float32),
        scratch_shapes=[
            pltpu.VMEM((NBUF, E_BLK, D_MODEL, D_MODEL), jnp.float32),
            pltpu.VMEM((NBUF, E_BLK, D_MODEL, D_MODEL), jnp.float32),
            pltpu.VMEM((NBUF, E_BLK, D_MODEL), jnp.float32),
            pltpu.VMEM((NBUF, E_BLK, D_MODEL), jnp.float32),
            pltpu.VMEM((N_TOKENS, D_MODEL), jnp.float32),
            pltpu.SemaphoreType.DMA((NBUF, 4)),
        ],
    )(x_flat, gate_w, w1, b1, w2, b2)
    return out.reshape(Bs, Ts, D)
